# Initial kernel scaffold; baseline (speedup 1.0000x reference)
import jax
import jax.numpy as jnp
from jax.experimental import pallas as pl


def kernel(x, W_g, W_s):
    raise NotImplementedError("write your pallas kernel here")


# speedup vs baseline: 1.2482x; 1.2482x over previous
import functools

import jax
import jax.numpy as jnp
from jax import lax
from jax.experimental import pallas as pl
from jax.experimental.pallas import tpu as pltpu
from jax.experimental.pallas import tpu_sc as plsc

VOCAB = 100000
D = 128
S = 8
LEN = 52
B = 4096

NC = 2
NS = 16
NW = NC * NS
C = B // NW
GROUP = 16
SUB = 2
N_GROUPS = C // GROUP
N_SUB = GROUP // SUB


def _token_compute(rows_v, ws_v, lt, wt, pos, out_v):
    iota = lax.iota(jnp.int32, 16)
    rb = lt * LEN

    def ctx_body(r, accs):
        row = rb + 2 + r
        return tuple(accs[c] + rows_v[row, pl.ds(16 * c, 16)] for c in range(8))

    ctx = lax.fori_loop(
        0, LEN - 2, ctx_body,
        tuple(jnp.zeros((16,), jnp.float32) for _ in range(8)),
        unroll=5)

    wrow = jnp.full((16,), wt, jnp.int32)

    scores = []
    for s in range(S):
        acc = jnp.zeros((16,), jnp.float32)
        for c in range(8):
            col = (16 * c + iota) * S + s
            g = plsc.load_gather(ws_v, [wrow, col])
            acc = acc + g * ctx[c]
        scores.append(jnp.sum(acc))

    best = scores[0]
    bs = jnp.int32(0)
    for s in range(1, S):
        gt = scores[s] > best
        best = jnp.where(gt, scores[s], best)
        bs = jnp.where(gt, jnp.int32(s), bs)

    acc = jnp.zeros((16,), jnp.float32)
    for c in range(8):
        col = (16 * c + iota) * S + bs
        ch = plsc.load_gather(ws_v, [wrow, col])
        acc = acc + ch * rows_v[rb + 1, pl.ds(16 * c, 16)]
    dot = jnp.sum(acc)

    sg = 1.0 / (1.0 + jnp.exp(jnp.full((16,), -dot, jnp.float32)))
    plsc.store_scatter(out_v, [jnp.full((16,), pos, jnp.int32)], sg,
                       mask=iota == 0)


def _body(xf_hbm, wg_hbm, ws_hbm, out_hbm,
          xidx_v, ws_v, rows_v, out_v, sem_x, sem_ws, sem_rows, sem_out):
    wid = lax.axis_index("s") * NC + lax.axis_index("c")
    tok0 = wid * C
    iota = lax.iota(jnp.int32, 16)

    xoff = pl.multiple_of(tok0 * LEN, 8)
    pltpu.async_copy(xf_hbm.at[pl.ds(xoff, C * LEN)], xidx_v, sem_x).wait()

    def group_body(g, _):
        tbase = g * GROUP
        x0idx = plsc.load_gather(xidx_v, [(tbase + iota) * LEN])
        pltpu.async_copy(ws_hbm.at[x0idx], ws_v, sem_ws).wait()

        def sub_body(h, _):
            t2 = tbase + SUB * h
            off = pl.multiple_of(t2 * LEN, 8)
            idxsl = xidx_v.at[pl.ds(off, SUB * LEN)]
            pltpu.async_copy(wg_hbm.at[idxsl], rows_v, sem_rows).wait()

            def tok_body(lt, _):
                _token_compute(rows_v, ws_v, lt, SUB * h + lt,
                               tbase + SUB * h + lt, out_v)
                return 0

            lax.fori_loop(0, SUB, tok_body, 0)
            return 0

        lax.fori_loop(0, N_SUB, sub_body, 0)
        return 0

    lax.fori_loop(0, N_GROUPS, group_body, 0)

    oof = pl.multiple_of(tok0, 8)
    pltpu.async_copy(out_v, out_hbm.at[pl.ds(oof, C)], sem_out).wait()


@functools.partial(jax.jit, static_argnames=())
def _run(xf, wg, wsf):
    mesh = plsc.VectorSubcoreMesh(core_axis_name="c", subcore_axis_name="s",
                                  num_cores=NC, num_subcores=NS)
    f = pl.kernel(
        _body,
        out_type=jax.ShapeDtypeStruct((B,), jnp.float32),
        mesh=mesh,
        scratch_types=[
            pltpu.VMEM((C * LEN,), jnp.int32),
            pltpu.VMEM((GROUP, D * S), jnp.float32),
            pltpu.VMEM((SUB * LEN, D), jnp.float32),
            pltpu.VMEM((C,), jnp.float32),
            pltpu.SemaphoreType.DMA,
            pltpu.SemaphoreType.DMA,
            pltpu.SemaphoreType.DMA,
            pltpu.SemaphoreType.DMA,
        ],
        compiler_params=pltpu.CompilerParams(needs_layout_passes=False),
    )
    return f(xf, wg, wsf)


def kernel(x, W_g, W_s):
    xf = x.reshape(-1)
    wsf = W_s.reshape(VOCAB, D * S)
    out = _run(xf, W_g, wsf)
    return out.reshape(B, 1)


# speedup vs baseline: 1.3428x; 1.0757x over previous
import functools

import jax
import jax.numpy as jnp
from jax import lax
from jax.experimental import pallas as pl
from jax.experimental.pallas import tpu as pltpu
from jax.experimental.pallas import tpu_sc as plsc

VOCAB = 100000
D = 128
S = 8
LEN = 52
B = 4096

NC = 2
NS = 16
NW = NC * NS
C = B // NW
GROUP = 16
SUB = 2
N_GROUPS = C // GROUP
N_SUB = GROUP // SUB


def _token_compute(rows_v, ws_v, lt, wt, pos, out_v):
    iota = lax.iota(jnp.int32, 16)
    rb = lt * LEN

    def ctx_body(r, accs):
        row = rb + 2 + r
        return tuple(accs[c] + rows_v[row, pl.ds(16 * c, 16)] for c in range(8))

    ctx = lax.fori_loop(
        0, LEN - 2, ctx_body,
        tuple(jnp.zeros((16,), jnp.float32) for _ in range(8)),
        unroll=5)

    wrow = jnp.full((16,), wt, jnp.int32)

    scores = []
    for s in range(S):
        acc = jnp.zeros((16,), jnp.float32)
        for c in range(8):
            col = (16 * c + iota) * S + s
            g = plsc.load_gather(ws_v, [wrow, col])
            acc = acc + g * ctx[c]
        scores.append(jnp.sum(acc))

    best = scores[0]
    bs = jnp.int32(0)
    for s in range(1, S):
        gt = scores[s] > best
        best = jnp.where(gt, scores[s], best)
        bs = jnp.where(gt, jnp.int32(s), bs)

    acc = jnp.zeros((16,), jnp.float32)
    for c in range(8):
        col = (16 * c + iota) * S + bs
        ch = plsc.load_gather(ws_v, [wrow, col])
        acc = acc + ch * rows_v[rb + 1, pl.ds(16 * c, 16)]
    dot = jnp.sum(acc)

    sg = 1.0 / (1.0 + jnp.exp(jnp.full((16,), -dot, jnp.float32)))
    plsc.store_scatter(out_v, [jnp.full((16,), pos, jnp.int32)], sg,
                       mask=iota == 0)


TOTAL_SUB = N_GROUPS * N_SUB


def _body(xf_hbm, wg_hbm, ws_hbm, out_hbm,
          xidx_v, ws0_v, ws1_v, rows0_v, rows1_v, out_v,
          sem_x, sem_ws0, sem_ws1, sem_rows0, sem_rows1, sem_out):
    wid = lax.axis_index("s") * NC + lax.axis_index("c")
    tok0 = wid * C
    iota = lax.iota(jnp.int32, 16)
    ws_bufs = (ws0_v, ws1_v)
    ws_sems = (sem_ws0, sem_ws1)
    rows_bufs = (rows0_v, rows1_v)
    rows_sems = (sem_rows0, sem_rows1)

    xoff = pl.multiple_of(tok0 * LEN, 8)
    pltpu.async_copy(xf_hbm.at[pl.ds(xoff, C * LEN)], xidx_v, sem_x).wait()

    def issue_ws(g, buf, sem):
        x0idx = plsc.load_gather(xidx_v, [(g * GROUP + iota) * LEN])
        pltpu.async_copy(ws_hbm.at[x0idx], buf, sem)

    def idxsl(j):
        off = pl.multiple_of(j * SUB * LEN, 8)
        return xidx_v.at[pl.ds(off, SUB * LEN)]

    def issue_rows(j, buf, sem):
        pltpu.async_copy(wg_hbm.at[idxsl(j)], buf, sem)

    issue_ws(0, ws_bufs[0], ws_sems[0])
    issue_rows(0, rows_bufs[0], rows_sems[0])

    def group2_body(gg, _):
        for p in range(2):
            g = 2 * gg + p
            pltpu.make_async_copy(ws_hbm.at[pl.ds(0, GROUP)],
                                  ws_bufs[p], ws_sems[p]).wait()

            @pl.when(g + 1 < N_GROUPS)
            def _():
                issue_ws(g + 1, ws_bufs[1 - p], ws_sems[1 - p])

            def sub2_body(hh, _):
                for q in range(2):
                    h = 2 * hh + q
                    j = g * N_SUB + h
                    pltpu.make_async_copy(
                        wg_hbm.at[pl.ds(0, SUB * LEN)],
                        rows_bufs[q], rows_sems[q]).wait()

                    @pl.when(j + 1 < TOTAL_SUB)
                    def _():
                        issue_rows(j + 1, rows_bufs[1 - q], rows_sems[1 - q])

                    def tok_body(lt, _):
                        _token_compute(rows_bufs[q], ws_bufs[p], lt,
                                       SUB * h + lt,
                                       g * GROUP + SUB * h + lt, out_v)
                        return 0

                    lax.fori_loop(0, SUB, tok_body, 0)
                return 0

            lax.fori_loop(0, N_SUB // 2, sub2_body, 0)
        return 0

    lax.fori_loop(0, N_GROUPS // 2, group2_body, 0)

    oof = pl.multiple_of(tok0, 8)
    pltpu.async_copy(out_v, out_hbm.at[pl.ds(oof, C)], sem_out).wait()


@functools.partial(jax.jit, static_argnames=())
def _run(xf, wg, wsf):
    mesh = plsc.VectorSubcoreMesh(core_axis_name="c", subcore_axis_name="s",
                                  num_cores=NC, num_subcores=NS)
    f = pl.kernel(
        _body,
        out_type=jax.ShapeDtypeStruct((B,), jnp.float32),
        mesh=mesh,
        scratch_types=[
            pltpu.VMEM((C * LEN,), jnp.int32),
            pltpu.VMEM((GROUP, D * S), jnp.float32),
            pltpu.VMEM((GROUP, D * S), jnp.float32),
            pltpu.VMEM((SUB * LEN, D), jnp.float32),
            pltpu.VMEM((SUB * LEN, D), jnp.float32),
            pltpu.VMEM((C,), jnp.float32),
            pltpu.SemaphoreType.DMA,
            pltpu.SemaphoreType.DMA,
            pltpu.SemaphoreType.DMA,
            pltpu.SemaphoreType.DMA,
            pltpu.SemaphoreType.DMA,
            pltpu.SemaphoreType.DMA,
        ],
        compiler_params=pltpu.CompilerParams(needs_layout_passes=False),
    )
    return f(xf, wg, wsf)


def kernel(x, W_g, W_s):
    xf = x.reshape(-1)
    wsf = W_s.reshape(VOCAB, D * S)
    out = _run(xf, W_g, wsf)
    return out.reshape(B, 1)


# speedup vs baseline: 12.9753x; 9.6632x over previous
import functools

import jax
import jax.numpy as jnp
from jax import lax
from jax.experimental import pallas as pl
from jax.experimental.pallas import tpu as pltpu
from jax.experimental.pallas import tpu_sc as plsc

VOCAB = 100000
D = 128
S = 8
LEN = 52
B = 4096

NC = 2
NS = 16
NW = NC * NS
C = B // NW
GROUP = 16
SUB = 2
N_GROUPS = C // GROUP
N_SUB = GROUP // SUB


def _token_compute(rows_v, ws_v, lt, wt, pos, out_v):
    iota = lax.iota(jnp.int32, 16)
    rb = lt * LEN

    def ctx_body(r, accs):
        row = rb + 2 + r
        return tuple(accs[c] + rows_v[row, pl.ds(16 * c, 16)] for c in range(8))

    ctx = lax.fori_loop(
        0, LEN - 2, ctx_body,
        tuple(jnp.zeros((16,), jnp.float32) for _ in range(8)),
        unroll=10)

    scores = []
    for s in range(S):
        acc = jnp.zeros((16,), jnp.float32)
        for c in range(8):
            acc = acc + ws_v[wt, s, pl.ds(16 * c, 16)] * ctx[c]
        scores.append(jnp.sum(acc))

    best = scores[0]
    bs = jnp.int32(0)
    for s in range(1, S):
        gt = scores[s] > best
        best = jnp.where(gt, scores[s], best)
        bs = jnp.where(gt, jnp.int32(s), bs)

    acc = jnp.zeros((16,), jnp.float32)
    for c in range(8):
        ch = ws_v[wt, bs, pl.ds(16 * c, 16)]
        acc = acc + ch * rows_v[rb + 1, pl.ds(16 * c, 16)]
    dot = jnp.sum(acc)

    sg = 1.0 / (1.0 + jnp.exp(jnp.full((16,), -dot, jnp.float32)))
    plsc.store_scatter(out_v, [jnp.full((16,), pos, jnp.int32)], sg,
                       mask=iota == 0)


TOTAL_SUB = N_GROUPS * N_SUB


def _body(xf_hbm, wg_hbm, ws_hbm, out_hbm,
          xidx_v, ws0_v, ws1_v, rows0_v, rows1_v, rows2_v, rows3_v, out_v,
          sem_x, sem_ws0, sem_ws1, sem_rows0, sem_rows1, sem_rows2,
          sem_rows3, sem_out):
    wid = lax.axis_index("s") * NC + lax.axis_index("c")
    tok0 = wid * C
    iota = lax.iota(jnp.int32, 16)
    ws_bufs = (ws0_v, ws1_v)
    ws_sems = (sem_ws0, sem_ws1)
    rows_bufs = (rows0_v, rows1_v, rows2_v, rows3_v)
    rows_sems = (sem_rows0, sem_rows1, sem_rows2, sem_rows3)

    xoff = pl.multiple_of(tok0 * LEN, 8)
    pltpu.async_copy(xf_hbm.at[pl.ds(xoff, C * LEN)], xidx_v, sem_x).wait()

    def issue_ws(g, buf, sem):
        x0idx = plsc.load_gather(xidx_v, [(g * GROUP + iota) * LEN])
        pltpu.async_copy(ws_hbm.at[x0idx], buf, sem)

    def idxsl(j):
        off = pl.multiple_of(j * SUB * LEN, 8)
        return xidx_v.at[pl.ds(off, SUB * LEN)]

    def issue_rows(j, buf, sem):
        pltpu.async_copy(wg_hbm.at[idxsl(j)], buf, sem)

    issue_ws(0, ws_bufs[0], ws_sems[0])
    for k in range(3):
        issue_rows(k, rows_bufs[k], rows_sems[k])

    def group2_body(gg, _):
        for p in range(2):
            g = 2 * gg + p
            pltpu.make_async_copy(ws_hbm.at[pl.ds(0, GROUP)],
                                  ws_bufs[p], ws_sems[p]).wait()

            @pl.when(g + 1 < N_GROUPS)
            def _():
                issue_ws(g + 1, ws_bufs[1 - p], ws_sems[1 - p])

            def sub4_body(hh, _):
                for q in range(4):
                    h = 4 * hh + q
                    j = g * N_SUB + h
                    pltpu.make_async_copy(
                        wg_hbm.at[pl.ds(0, SUB * LEN)],
                        rows_bufs[q], rows_sems[q]).wait()

                    @pl.when(j + 3 < TOTAL_SUB)
                    def _():
                        nq = (q + 3) % 4
                        issue_rows(j + 3, rows_bufs[nq], rows_sems[nq])

                    def tok_body(lt, _):
                        _token_compute(rows_bufs[q], ws_bufs[p], lt,
                                       SUB * h + lt,
                                       g * GROUP + SUB * h + lt, out_v)
                        return 0

                    lax.fori_loop(0, SUB, tok_body, 0)
                return 0

            lax.fori_loop(0, N_SUB // 4, sub4_body, 0)
        return 0

    lax.fori_loop(0, N_GROUPS // 2, group2_body, 0)

    oof = pl.multiple_of(tok0, 8)
    pltpu.async_copy(out_v, out_hbm.at[pl.ds(oof, C)], sem_out).wait()


@functools.partial(jax.jit, static_argnames=())
def _run(xf, wg, wsf):
    mesh = plsc.VectorSubcoreMesh(core_axis_name="c", subcore_axis_name="s",
                                  num_cores=NC, num_subcores=NS)
    f = pl.kernel(
        _body,
        out_type=jax.ShapeDtypeStruct((B,), jnp.float32),
        mesh=mesh,
        scratch_types=[
            pltpu.VMEM((C * LEN,), jnp.int32),
            pltpu.VMEM((GROUP, S, D), jnp.float32),
            pltpu.VMEM((GROUP, S, D), jnp.float32),
            pltpu.VMEM((SUB * LEN, D), jnp.float32),
            pltpu.VMEM((SUB * LEN, D), jnp.float32),
            pltpu.VMEM((SUB * LEN, D), jnp.float32),
            pltpu.VMEM((SUB * LEN, D), jnp.float32),
            pltpu.VMEM((C,), jnp.float32),
            pltpu.SemaphoreType.DMA,
            pltpu.SemaphoreType.DMA,
            pltpu.SemaphoreType.DMA,
            pltpu.SemaphoreType.DMA,
            pltpu.SemaphoreType.DMA,
            pltpu.SemaphoreType.DMA,
            pltpu.SemaphoreType.DMA,
            pltpu.SemaphoreType.DMA,
        ],
        compiler_params=pltpu.CompilerParams(needs_layout_passes=False),
    )
    return f(xf, wg, wsf)


def kernel(x, W_g, W_s):
    xf = x.reshape(-1)
    wst = jnp.transpose(W_s, (0, 2, 1))
    out = _run(xf, W_g, wst)
    return out.reshape(B, 1)
